# baseline (speedup 1.0000x reference)
import functools

import jax
import jax.numpy as jnp
from jax import lax
from jax.experimental import pallas as pl
from jax.experimental.pallas import tpu as pltpu
from jax.experimental.pallas import tpu_sc as plsc

DIM = 128
NC, NS = 2, 16
NW = NC * NS
NTOK = 4096
SEQ = 50
SEQP = 56
TPW = NTOK // NW

_mesh = plsc.VectorSubcoreMesh(core_axis_name="c", subcore_axis_name="s")


@functools.partial(
    pl.kernel,
    mesh=_mesh,
    out_type=jax.ShapeDtypeStruct((NTOK, SEQ, DIM), jnp.float32),
    scratch_types=[
        pltpu.VMEM((SEQP, TPW), jnp.int32),
        pltpu.VMEM((TPW, DIM), jnp.float32),
        pltpu.VMEM((TPW, DIM), jnp.float32),
        pltpu.SemaphoreType.DMA,
        pltpu.SemaphoreType.DMA,
        pltpu.SemaphoreType.DMA,
        pltpu.SemaphoreType.DMA,
    ],
)
def _gather_kernel(table, idx_hbm, out, idx_v, buf_a, buf_b,
                   in_a, in_b, out_a, out_b):
    wid = lax.axis_index("s") * NC + lax.axis_index("c")
    s0 = wid * TPW
    pltpu.sync_copy(idx_hbm.at[:, pl.ds(s0, TPW)], idx_v)

    def gstart(q, buf, sem):
        pltpu.async_copy(table.at[idx_v.at[q]], buf, sem)

    def gwait(buf, sem):
        pltpu.make_async_copy(table.at[pl.ds(0, TPW)], buf, sem).wait()

    def wstart(q, buf, sem):
        pltpu.async_copy(buf, out.at[pl.ds(s0, TPW), q], sem)

    def wwait(buf, sem):
        pltpu.make_async_copy(buf, out.at[pl.ds(s0, TPW), 0], sem).wait()

    gstart(0, buf_a, in_a)
    gstart(1, buf_b, in_b)

    def body(i, carry):
        q0 = 2 * i
        gwait(buf_a, in_a)
        wstart(q0, buf_a, out_a)
        gwait(buf_b, in_b)
        wstart(q0 + 1, buf_b, out_b)
        wwait(buf_a, out_a)
        gstart(q0 + 2, buf_a, in_a)
        wwait(buf_b, out_b)
        gstart(q0 + 3, buf_b, in_b)
        return carry

    lax.fori_loop(0, (SEQ - 2) // 2, body, 0)

    gwait(buf_a, in_a)
    wstart(SEQ - 2, buf_a, out_a)
    gwait(buf_b, in_b)
    wstart(SEQ - 1, buf_b, out_b)
    wwait(buf_a, out_a)
    wwait(buf_b, out_b)


def kernel(token_ids, embeddings):
    ids_t = token_ids.astype(jnp.int32).T
    idx = jnp.pad(ids_t, ((0, SEQP - SEQ), (0, 0)))
    return _gather_kernel(embeddings, idx)


# speedup vs baseline: 1.0185x; 1.0185x over previous
import functools

import jax
import jax.numpy as jnp
from jax import lax
from jax.experimental import pallas as pl
from jax.experimental.pallas import tpu as pltpu
from jax.experimental.pallas import tpu_sc as plsc

DIM = 128
NC, NS = 2, 16
NW = NC * NS
NTOK = 4096
SEQ = 50
IPAD = 64
TPW = NTOK // NW
G = 4
NGRP = TPW // G

_mesh = plsc.VectorSubcoreMesh(core_axis_name="c", subcore_axis_name="s")


@functools.partial(
    pl.kernel,
    mesh=_mesh,
    out_type=jax.ShapeDtypeStruct((NTOK, SEQ, DIM), jnp.float32),
    scratch_types=[
        pltpu.VMEM((TPW * IPAD,), jnp.int32),
        pltpu.VMEM((G, SEQ, DIM), jnp.float32),
        pltpu.VMEM((G, SEQ, DIM), jnp.float32),
        pltpu.SemaphoreType.DMA,
        pltpu.SemaphoreType.DMA,
        pltpu.SemaphoreType.DMA,
        pltpu.SemaphoreType.DMA,
    ],
)
def _gather_kernel(table, idx_hbm, out, idx_v, buf_a, buf_b,
                   in_a, in_b, out_a, out_b):
    wid = lax.axis_index("s") * NC + lax.axis_index("c")
    s0 = wid * TPW
    pltpu.sync_copy(idx_hbm.at[pl.ds(s0 * IPAD, TPW * IPAD)], idx_v)

    def gstart(g, buf, sem):
        for j in range(G):
            pltpu.async_copy(
                table.at[idx_v.at[pl.ds((g * G + j) * IPAD, SEQ)]],
                buf.at[j], sem)

    def gwait(buf, sem):
        for j in range(G):
            pltpu.make_async_copy(out.at[0], buf.at[j], sem).wait()

    def wstart(g, buf, sem):
        pltpu.async_copy(buf, out.at[pl.ds(s0 + g * G, G)], sem)

    def wwait(buf, sem):
        pltpu.make_async_copy(buf, out.at[pl.ds(s0, G)], sem).wait()

    gstart(0, buf_a, in_a)
    gstart(1, buf_b, in_b)

    def body(i, carry):
        g0 = 2 * i
        gwait(buf_a, in_a)
        wstart(g0, buf_a, out_a)
        gwait(buf_b, in_b)
        wstart(g0 + 1, buf_b, out_b)
        wwait(buf_a, out_a)
        gstart(g0 + 2, buf_a, in_a)
        wwait(buf_b, out_b)
        gstart(g0 + 3, buf_b, in_b)
        return carry

    lax.fori_loop(0, (NGRP - 2) // 2, body, 0)

    gwait(buf_a, in_a)
    wstart(NGRP - 2, buf_a, out_a)
    gwait(buf_b, in_b)
    wstart(NGRP - 1, buf_b, out_b)
    wwait(buf_a, out_a)
    wwait(buf_b, out_b)


def kernel(token_ids, embeddings):
    ids = token_ids.astype(jnp.int32)
    idx = jnp.pad(ids, ((0, 0), (0, IPAD - SEQ))).reshape(-1)
    return _gather_kernel(embeddings, idx)
